# baseline (speedup 1.0000x reference)
import functools

import jax
import jax.numpy as jnp
from jax import lax
from jax.experimental import pallas as pl
from jax.experimental.pallas import tpu as pltpu
from jax.experimental.pallas import tpu_sc as plsc

_DECAY = 1e-4
_EPS = 0.03

_NC = 2
_NS = 16
_NW = _NC * _NS


def _sc_gather(user_embed, item_embed, noise_u, noise_i, users, pos_items,
               neg_flat):
  B = users.shape[0]
  D = user_embed.shape[1]
  BK = neg_flat.shape[0]
  per_w = B // _NW
  negs_per_w = BK // _NW
  neg_chunk = min(negs_per_w, 2048)
  n_chunks = negs_per_w // neg_chunk

  mesh = plsc.VectorSubcoreMesh(core_axis_name="c", subcore_axis_name="s",
                                num_cores=_NC)
  f32 = jnp.float32

  @functools.partial(
      pl.kernel,
      out_type=[
          jax.ShapeDtypeStruct((B, D), f32),
          jax.ShapeDtypeStruct((B, D), f32),
          jax.ShapeDtypeStruct((B, D), f32),
          jax.ShapeDtypeStruct((B, D), f32),
          jax.ShapeDtypeStruct((BK, D), f32),
      ],
      mesh=mesh,
      scratch_types=[
          pltpu.VMEM((per_w,), jnp.int32),
          pltpu.VMEM((per_w, D), f32),
          pltpu.VMEM((neg_chunk,), jnp.int32),
          pltpu.VMEM((neg_chunk, D), f32),
          pltpu.SemaphoreType.DMA,
      ],
      compiler_params=pltpu.CompilerParams(use_tc_tiling_on_sc=False),
  )
  def gather_kernel(user_hbm, item_hbm, nu_hbm, ni_hbm, users_hbm, pos_hbm,
                    neg_hbm, u_out, pos_out, nu_out, ni_out, neg_out,
                    idx_v, rows_v, nidx_v, nrows_v, sem):
    wid = lax.axis_index("s") * _NC + lax.axis_index("c")
    base = wid * per_w
    pltpu.sync_copy(users_hbm.at[pl.ds(base, per_w)], idx_v)
    pltpu.async_copy(user_hbm.at[idx_v], rows_v, sem).wait()
    pltpu.sync_copy(rows_v, u_out.at[pl.ds(base, per_w)])
    pltpu.async_copy(nu_hbm.at[idx_v], rows_v, sem).wait()
    pltpu.sync_copy(rows_v, nu_out.at[pl.ds(base, per_w)])
    pltpu.sync_copy(pos_hbm.at[pl.ds(base, per_w)], idx_v)
    pltpu.async_copy(item_hbm.at[idx_v], rows_v, sem).wait()
    pltpu.sync_copy(rows_v, pos_out.at[pl.ds(base, per_w)])
    pltpu.async_copy(ni_hbm.at[idx_v], rows_v, sem).wait()
    pltpu.sync_copy(rows_v, ni_out.at[pl.ds(base, per_w)])
    nbase = wid * negs_per_w
    for t in range(n_chunks):
      off = nbase + t * neg_chunk
      pltpu.sync_copy(neg_hbm.at[pl.ds(off, neg_chunk)], nidx_v)
      pltpu.async_copy(item_hbm.at[nidx_v], nrows_v, sem).wait()
      pltpu.sync_copy(nrows_v, neg_out.at[pl.ds(off, neg_chunk)])

  return gather_kernel(user_embed, item_embed, noise_u, noise_i, users,
                       pos_items, neg_flat)


def _tc_score(u_rows, pos_rows, nu_rows, ni_rows, neg_rows, B, K, D):
  Bb = min(512, B)
  grid = B // Bb
  f32 = jnp.float32

  def body(u_ref, pos_ref, nu_ref, ni_ref, neg_ref, ypos_ref, yneg_ref,
           loss_ref, clu_ref, cli_ref):
    i = pl.program_id(0)
    u = u_ref[...]
    pos = pos_ref[...]
    nu = nu_ref[...]
    ni = ni_ref[...]
    neg = neg_ref[...]

    def inv_norm(x):
      n2 = jnp.sum(x * x, axis=-1, keepdims=True)
      return 1.0 / jnp.maximum(jnp.sqrt(n2), 1e-12)

    u_n = u * inv_norm(u)
    ypos_ref[...] = jnp.sum(pos * u_n, axis=-1, keepdims=True) * inv_norm(pos)
    neg_dot = jnp.sum(neg * u_n[:, None, :], axis=-1)
    neg_n2 = jnp.sum(neg * neg, axis=-1)
    yneg_ref[...] = neg_dot / jnp.maximum(jnp.sqrt(neg_n2), 1e-12)

    clu_ref[...] = u + jnp.sign(u) * (nu * inv_norm(nu)) * _EPS
    cli_ref[...] = pos + jnp.sign(pos) * (ni * inv_norm(ni)) * _EPS

    part = jnp.sum(u * u) + jnp.sum(pos * pos) + jnp.sum(neg * neg)

    @pl.when(i == 0)
    def _():
      loss_ref[...] = jnp.zeros((1, 1), jnp.float32)

    loss_ref[...] = loss_ref[...] + part

    @pl.when(i == pl.num_programs(0) - 1)
    def _():
      loss_ref[...] = loss_ref[...] * (_DECAY / (2.0 * B))

  return pl.pallas_call(
      body,
      grid=(grid,),
      in_specs=[
          pl.BlockSpec((Bb, D), lambda i: (i, 0)),
          pl.BlockSpec((Bb, D), lambda i: (i, 0)),
          pl.BlockSpec((Bb, D), lambda i: (i, 0)),
          pl.BlockSpec((Bb, D), lambda i: (i, 0)),
          pl.BlockSpec((Bb, K, D), lambda i: (i, 0, 0)),
      ],
      out_specs=[
          pl.BlockSpec((Bb, 1), lambda i: (i, 0)),
          pl.BlockSpec((Bb, K), lambda i: (i, 0)),
          pl.BlockSpec((1, 1), lambda i: (0, 0)),
          pl.BlockSpec((Bb, D), lambda i: (i, 0)),
          pl.BlockSpec((Bb, D), lambda i: (i, 0)),
      ],
      out_shape=[
          jax.ShapeDtypeStruct((B, 1), f32),
          jax.ShapeDtypeStruct((B, K), f32),
          jax.ShapeDtypeStruct((1, 1), f32),
          jax.ShapeDtypeStruct((B, D), f32),
          jax.ShapeDtypeStruct((B, D), f32),
      ],
  )(u_rows, pos_rows, nu_rows, ni_rows, neg_rows)


def kernel(user_embed, item_embed, noise_u, noise_i, users, pos_items,
           neg_items):
  B = users.shape[0]
  K = neg_items.shape[1]
  D = user_embed.shape[1]

  users = users.astype(jnp.int32)
  pos_items = pos_items.astype(jnp.int32)
  neg_flat = neg_items.astype(jnp.int32).reshape(B * K)

  u_rows, pos_rows, nu_rows, ni_rows, neg_flat_rows = _sc_gather(
      user_embed, item_embed, noise_u, noise_i, users, pos_items, neg_flat)

  neg_rows = neg_flat_rows.reshape(B, K, D)
  ypos, yneg, loss, cl_u_e, cl_i_e = _tc_score(
      u_rows, pos_rows, nu_rows, ni_rows, neg_rows, B, K, D)

  y_pred = jnp.concatenate([ypos, yneg], axis=1)
  return (y_pred, loss[0, 0], cl_u_e, cl_i_e)
